# Initial kernel scaffold; baseline (speedup 1.0000x reference)
import functools

import jax
import jax.numpy as jnp
from jax import lax
from jax.experimental import pallas as pl
from jax.experimental.pallas import tpu as pltpu
from jax.experimental.pallas import tpu_sc as plsc

_NC, _NS, _L = 2, 16, 16
_NW = _NC * _NS
_NP = 10240


def _deg_partials(dst, np_pad):
    e = dst.shape[0]
    per_tile = e // _NW
    bsz = 128
    full, tail = divmod(per_tile, bsz)
    rpt = np_pad // _NS
    mesh = plsc.VectorSubcoreMesh(core_axis_name="c", subcore_axis_name="s")

    @functools.partial(
        pl.kernel,
        out_type=jax.ShapeDtypeStruct((_NC * np_pad,), jnp.float32),
        mesh=mesh,
        scratch_types=[
            pltpu.VMEM((bsz,), jnp.int32),
            pltpu.VMEM((bsz,), jnp.float32),
            pltpu.VMEM((tail,), jnp.int32),
            pltpu.VMEM((tail,), jnp.float32),
            pltpu.VMEM((rpt,), jnp.float32),
            pltpu.VMEM_SHARED((np_pad,), jnp.float32),
        ],
    )
    def k(dst_hbm, out_hbm, didx, ones_v, didx_t, ones_t, zeros_v, acc):
        c = lax.axis_index("c")
        s = lax.axis_index("s")
        wid = c * _NS + s
        for j in range(bsz // _L):
            ones_v[pl.ds(j * _L, _L)] = jnp.full((_L,), 1.0, jnp.float32)
        for j in range(tail // _L):
            ones_t[pl.ds(j * _L, _L)] = jnp.full((_L,), 1.0, jnp.float32)
        for j in range(rpt // _L):
            zeros_v[pl.ds(j * _L, _L)] = jnp.zeros((_L,), jnp.float32)
        pltpu.sync_copy(zeros_v, acc.at[pl.ds(s * rpt, rpt)])
        plsc.subcore_barrier()
        base = wid * per_tile

        def body(i, carry):
            off = base + i * bsz
            pltpu.sync_copy(dst_hbm.at[pl.ds(off, bsz)], didx)
            pltpu.sync_copy(ones_v, acc.at[didx], add=True)
            return carry

        lax.fori_loop(0, full, body, 0)
        if tail:
            off = base + full * bsz
            pltpu.sync_copy(dst_hbm.at[pl.ds(off, tail)], didx_t)
            pltpu.sync_copy(ones_t, acc.at[didx_t], add=True)
        plsc.subcore_barrier()
        pltpu.sync_copy(acc.at[pl.ds(s * rpt, rpt)],
                        out_hbm.at[pl.ds(c * np_pad + s * rpt, rpt)])

    return k(dst)


def _agg_partials(tbl, src, dst, np_pad, d):
    e = src.shape[0]
    per_tile = e // _NW
    bsz = 128
    full, tail = divmod(per_tile, bsz)
    rpt = np_pad // _NS
    zr = 16
    mesh = plsc.VectorSubcoreMesh(core_axis_name="c", subcore_axis_name="s")

    @functools.partial(
        pl.kernel,
        out_type=jax.ShapeDtypeStruct((_NC * np_pad, d), jnp.float32),
        mesh=mesh,
        scratch_types=[
            pltpu.VMEM((bsz,), jnp.int32),
            pltpu.VMEM((bsz,), jnp.int32),
            pltpu.VMEM((bsz, d), jnp.float32),
            pltpu.VMEM((tail,), jnp.int32),
            pltpu.VMEM((tail,), jnp.int32),
            pltpu.VMEM((tail, d), jnp.float32),
            pltpu.VMEM((zr, d), jnp.float32),
            pltpu.VMEM_SHARED((np_pad, d), jnp.float32),
            pltpu.SemaphoreType.DMA,
        ],
    )
    def k(tbl_hbm, src_hbm, dst_hbm, out_hbm,
          sidx, didx, rows, sidx_t, didx_t, rows_t, zblk, acc, sem):
        c = lax.axis_index("c")
        s = lax.axis_index("s")
        wid = c * _NS + s
        for r in range(zr):
            for j in range(d // _L):
                zblk[r, pl.ds(j * _L, _L)] = jnp.zeros((_L,), jnp.float32)
        for t in range(rpt // zr):
            pltpu.sync_copy(zblk, acc.at[pl.ds(s * rpt + t * zr, zr)])
        plsc.subcore_barrier()
        base = wid * per_tile

        def body(i, carry):
            off = base + i * bsz
            pltpu.sync_copy(src_hbm.at[pl.ds(off, bsz)], sidx)
            pltpu.sync_copy(dst_hbm.at[pl.ds(off, bsz)], didx)
            pltpu.async_copy(tbl_hbm.at[sidx], rows, sem).wait()
            pltpu.sync_copy(rows, acc.at[didx], add=True)
            return carry

        lax.fori_loop(0, full, body, 0)
        if tail:
            off = base + full * bsz
            pltpu.sync_copy(src_hbm.at[pl.ds(off, tail)], sidx_t)
            pltpu.sync_copy(dst_hbm.at[pl.ds(off, tail)], didx_t)
            pltpu.async_copy(tbl_hbm.at[sidx_t], rows_t, sem).wait()
            pltpu.sync_copy(rows_t, acc.at[didx_t], add=True)
        plsc.subcore_barrier()
        pltpu.sync_copy(acc.at[pl.ds(s * rpt, rpt)],
                        out_hbm.at[pl.ds(c * np_pad + s * rpt, rpt)])

    return k(tbl, src, dst)


def _tc_prep1(degp_t, x_pad, w1):
    np_pad, d = x_pad.shape
    blk = 1280
    grid = np_pad // blk

    def body(deg_ref, x_ref, w_ref, o_ref):
        dval = deg_ref[:, 0:1] + deg_ref[:, 1:2] + 1.0
        dis = lax.rsqrt(dval)
        o_ref[...] = jnp.dot(x_ref[...], w_ref[...],
                             preferred_element_type=jnp.float32) * dis

    return pl.pallas_call(
        body,
        grid=(grid,),
        in_specs=[
            pl.BlockSpec((blk, 2), lambda i: (i, 0)),
            pl.BlockSpec((blk, d), lambda i: (i, 0)),
            pl.BlockSpec((d, d), lambda i: (0, 0)),
        ],
        out_specs=pl.BlockSpec((blk, d), lambda i: (i, 0)),
        out_shape=jax.ShapeDtypeStruct((np_pad, d), jnp.float32),
    )(degp_t, x_pad, w1)


def _tc_layer2(p, t1, degp_t, b1, wcat):
    np_pad, d = t1.shape
    d2 = wcat.shape[1]
    blk = 1280
    grid = np_pad // blk

    def body(p_ref, t1_ref, deg_ref, b_ref, w_ref, o_ref):
        dval = deg_ref[:, 0:1] + deg_ref[:, 1:2] + 1.0
        dis = lax.rsqrt(dval)
        agg = p_ref[0] + p_ref[1] + t1_ref[...]
        h = jnp.maximum(agg * dis + b_ref[...], 0.0)
        o_ref[...] = jnp.dot(h, w_ref[...],
                             preferred_element_type=jnp.float32) * dis

    return pl.pallas_call(
        body,
        grid=(grid,),
        in_specs=[
            pl.BlockSpec((2, blk, d), lambda i: (0, i, 0)),
            pl.BlockSpec((blk, d), lambda i: (i, 0)),
            pl.BlockSpec((blk, 2), lambda i: (i, 0)),
            pl.BlockSpec((1, d), lambda i: (0, 0)),
            pl.BlockSpec((d, d2), lambda i: (0, 0)),
        ],
        out_specs=pl.BlockSpec((blk, d2), lambda i: (i, 0)),
        out_shape=jax.ShapeDtypeStruct((np_pad, d2), jnp.float32),
    )(p, t1, degp_t, b1, wcat)


def _tc_final(q, t2, degp_t, bcat):
    np_pad, d2 = t2.shape
    blk = 1280
    grid = np_pad // blk

    def body(q_ref, t2_ref, deg_ref, b_ref, o_ref):
        dval = deg_ref[:, 0:1] + deg_ref[:, 1:2] + 1.0
        dis = lax.rsqrt(dval)
        o_ref[...] = (q_ref[0] + q_ref[1] + t2_ref[...]) * dis + b_ref[...]

    return pl.pallas_call(
        body,
        grid=(grid,),
        in_specs=[
            pl.BlockSpec((2, blk, d2), lambda i: (0, i, 0)),
            pl.BlockSpec((blk, d2), lambda i: (i, 0)),
            pl.BlockSpec((blk, 2), lambda i: (i, 0)),
            pl.BlockSpec((1, d2), lambda i: (0, 0)),
        ],
        out_specs=pl.BlockSpec((blk, d2), lambda i: (i, 0)),
        out_shape=jax.ShapeDtypeStruct((np_pad, d2), jnp.float32),
    )(q, t2, degp_t, bcat)


def kernel(x, edge_index, W1, b1, W2, b2, Wd, bd):
    n, d = x.shape
    src = edge_index[0]
    dst = edge_index[1]
    nc = W2.shape[1]
    nd = Wd.shape[1]
    d2 = 48

    x_pad = jnp.zeros((_NP, d), x.dtype).at[:n].set(x)
    wcat = jnp.zeros((d, d2), W2.dtype).at[:, :nc].set(W2).at[:, nc:nc + nd].set(Wd)
    bcat = jnp.zeros((1, d2), b2.dtype).at[0, :nc].set(b2).at[0, nc:nc + nd].set(bd)

    degp_t = _deg_partials(dst, _NP).reshape(_NC, _NP).T
    t1 = _tc_prep1(degp_t, x_pad, W1)
    p = _agg_partials(t1, src, dst, _NP, d).reshape(_NC, _NP, d)
    t2 = _tc_layer2(p, t1, degp_t, b1.reshape(1, d), wcat)
    q = _agg_partials(t2, src, dst, _NP, d2).reshape(_NC, _NP, d2)
    out = _tc_final(q, t2, degp_t, bcat)
    return out[:n, :nc], out[:n, nc:nc + nd]


# speedup vs baseline: 19.1150x; 19.1150x over previous
import functools

import jax
import jax.numpy as jnp
from jax import lax
from jax.experimental import pallas as pl
from jax.experimental.pallas import tpu as pltpu
from jax.experimental.pallas import tpu_sc as plsc

_NC, _NS, _L = 2, 16, 16
_NW = _NC * _NS
_NP = 10240


def _deg_partials(dst, np_pad):
    e = dst.shape[0]
    per_tile = e // _NW
    bsz = 128
    full, tail = divmod(per_tile, bsz)
    rpt = np_pad // _NS
    mesh = plsc.VectorSubcoreMesh(core_axis_name="c", subcore_axis_name="s")

    @functools.partial(
        pl.kernel,
        out_type=jax.ShapeDtypeStruct((_NC * np_pad,), jnp.float32),
        mesh=mesh,
        scratch_types=[
            pltpu.VMEM((bsz,), jnp.int32),
            pltpu.VMEM((bsz,), jnp.float32),
            pltpu.VMEM((tail,), jnp.int32),
            pltpu.VMEM((tail,), jnp.float32),
            pltpu.VMEM((rpt,), jnp.float32),
            pltpu.VMEM_SHARED((np_pad,), jnp.float32),
        ],
    )
    def k(dst_hbm, out_hbm, didx, ones_v, didx_t, ones_t, zeros_v, acc):
        c = lax.axis_index("c")
        s = lax.axis_index("s")
        wid = c * _NS + s
        for j in range(bsz // _L):
            ones_v[pl.ds(j * _L, _L)] = jnp.full((_L,), 1.0, jnp.float32)
        for j in range(tail // _L):
            ones_t[pl.ds(j * _L, _L)] = jnp.full((_L,), 1.0, jnp.float32)
        for j in range(rpt // _L):
            zeros_v[pl.ds(j * _L, _L)] = jnp.zeros((_L,), jnp.float32)
        pltpu.sync_copy(zeros_v, acc.at[pl.ds(s * rpt, rpt)])
        plsc.subcore_barrier()
        base = wid * per_tile

        def body(i, carry):
            off = base + i * bsz
            pltpu.sync_copy(dst_hbm.at[pl.ds(off, bsz)], didx)
            pltpu.sync_copy(ones_v, acc.at[didx], add=True)
            return carry

        lax.fori_loop(0, full, body, 0)
        if tail:
            off = base + full * bsz
            pltpu.sync_copy(dst_hbm.at[pl.ds(off, tail)], didx_t)
            pltpu.sync_copy(ones_t, acc.at[didx_t], add=True)
        plsc.subcore_barrier()
        pltpu.sync_copy(acc.at[pl.ds(s * rpt, rpt)],
                        out_hbm.at[pl.ds(c * np_pad + s * rpt, rpt)])

    return k(dst)


def _agg_partials(tbl, src, dst, np_pad, d):
    e = src.shape[0]
    per_tile = e // _NW
    bsz = 128
    full, tail = divmod(per_tile, bsz)
    rpt = np_pad // _NS
    zr = 16
    mesh = plsc.VectorSubcoreMesh(core_axis_name="c", subcore_axis_name="s")

    @functools.partial(
        pl.kernel,
        out_type=jax.ShapeDtypeStruct((_NC * np_pad, d), jnp.float32),
        mesh=mesh,
        scratch_types=[
            pltpu.VMEM((bsz,), jnp.int32),
            pltpu.VMEM((bsz,), jnp.int32),
            pltpu.VMEM((bsz, d), jnp.float32),
            pltpu.VMEM((tail,), jnp.int32),
            pltpu.VMEM((tail,), jnp.int32),
            pltpu.VMEM((tail, d), jnp.float32),
            pltpu.VMEM((zr, d), jnp.float32),
            pltpu.VMEM_SHARED((np_pad, d), jnp.float32),
            pltpu.SemaphoreType.DMA,
        ],
    )
    def k(tbl_hbm, src_hbm, dst_hbm, out_hbm,
          sidx, didx, rows, sidx_t, didx_t, rows_t, zblk, acc, sem):
        c = lax.axis_index("c")
        s = lax.axis_index("s")
        wid = c * _NS + s
        for r in range(zr):
            for j in range(d // _L):
                zblk[r, pl.ds(j * _L, _L)] = jnp.zeros((_L,), jnp.float32)
        for t in range(rpt // zr):
            pltpu.sync_copy(zblk, acc.at[pl.ds(s * rpt + t * zr, zr)])
        plsc.subcore_barrier()
        base = wid * per_tile

        def body(i, carry):
            off = base + i * bsz
            pltpu.sync_copy(src_hbm.at[pl.ds(off, bsz)], sidx)
            pltpu.sync_copy(dst_hbm.at[pl.ds(off, bsz)], didx)
            pltpu.async_copy(tbl_hbm.at[sidx], rows, sem).wait()
            pltpu.sync_copy(rows, acc.at[didx], add=True)
            return carry

        lax.fori_loop(0, full, body, 0)
        if tail:
            off = base + full * bsz
            pltpu.sync_copy(src_hbm.at[pl.ds(off, tail)], sidx_t)
            pltpu.sync_copy(dst_hbm.at[pl.ds(off, tail)], didx_t)
            pltpu.async_copy(tbl_hbm.at[sidx_t], rows_t, sem).wait()
            pltpu.sync_copy(rows_t, acc.at[didx_t], add=True)
        plsc.subcore_barrier()
        pltpu.sync_copy(acc.at[pl.ds(s * rpt, rpt)],
                        out_hbm.at[pl.ds(c * np_pad + s * rpt, rpt)])

    return k(tbl, src, dst)


def _tc_prep1(degp_t, x_pad, w1):
    np_pad, d = x_pad.shape
    blk = 1280
    grid = np_pad // blk

    def body(deg_ref, x_ref, w_ref, o_ref):
        dval = deg_ref[:, 0:1] + deg_ref[:, 1:2] + 1.0
        dis = lax.rsqrt(dval)
        o_ref[...] = jnp.dot(x_ref[...], w_ref[...],
                             preferred_element_type=jnp.float32) * dis

    return pl.pallas_call(
        body,
        grid=(grid,),
        in_specs=[
            pl.BlockSpec((blk, 2), lambda i: (i, 0)),
            pl.BlockSpec((blk, d), lambda i: (i, 0)),
            pl.BlockSpec((d, d), lambda i: (0, 0)),
        ],
        out_specs=pl.BlockSpec((blk, d), lambda i: (i, 0)),
        out_shape=jax.ShapeDtypeStruct((np_pad, d), jnp.float32),
    )(degp_t, x_pad, w1)


def _tc_layer2(p, t1, degp_t, b1, wcat):
    np_pad, d = t1.shape
    d2 = wcat.shape[1]
    blk = 1280
    grid = np_pad // blk

    def body(p_ref, t1_ref, deg_ref, b_ref, w_ref, o_ref):
        dval = deg_ref[:, 0:1] + deg_ref[:, 1:2] + 1.0
        dis = lax.rsqrt(dval)
        agg = p_ref[0] + p_ref[1] + t1_ref[...]
        h = jnp.maximum(agg * dis + b_ref[...], 0.0)
        o_ref[...] = jnp.dot(h, w_ref[...],
                             preferred_element_type=jnp.float32) * dis

    return pl.pallas_call(
        body,
        grid=(grid,),
        in_specs=[
            pl.BlockSpec((2, blk, d), lambda i: (0, i, 0)),
            pl.BlockSpec((blk, d), lambda i: (i, 0)),
            pl.BlockSpec((blk, 2), lambda i: (i, 0)),
            pl.BlockSpec((1, d), lambda i: (0, 0)),
            pl.BlockSpec((d, d2), lambda i: (0, 0)),
        ],
        out_specs=pl.BlockSpec((blk, d2), lambda i: (i, 0)),
        out_shape=jax.ShapeDtypeStruct((np_pad, d2), jnp.float32),
    )(p, t1, degp_t, b1, wcat)


def _tc_final(q, t2, degp_t, bcat):
    np_pad, d2 = t2.shape
    blk = 1280
    grid = np_pad // blk

    def body(q_ref, t2_ref, deg_ref, b_ref, o_ref):
        dval = deg_ref[:, 0:1] + deg_ref[:, 1:2] + 1.0
        dis = lax.rsqrt(dval)
        o_ref[...] = (q_ref[0] + q_ref[1] + t2_ref[...]) * dis + b_ref[...]

    return pl.pallas_call(
        body,
        grid=(grid,),
        in_specs=[
            pl.BlockSpec((2, blk, d2), lambda i: (0, i, 0)),
            pl.BlockSpec((blk, d2), lambda i: (i, 0)),
            pl.BlockSpec((blk, 2), lambda i: (i, 0)),
            pl.BlockSpec((1, d2), lambda i: (0, 0)),
        ],
        out_specs=pl.BlockSpec((blk, d2), lambda i: (i, 0)),
        out_shape=jax.ShapeDtypeStruct((np_pad, d2), jnp.float32),
    )(q, t2, degp_t, bcat)


def kernel(x, edge_index, W1, b1, W2, b2, Wd, bd):
    n, d = x.shape
    src = edge_index[0]
    dst = edge_index[1]
    nc = W2.shape[1]
    nd = Wd.shape[1]
    d2 = 128

    x_pad = jnp.zeros((_NP, d), x.dtype).at[:n].set(x)
    wcat = jnp.zeros((d, d2), W2.dtype).at[:, :nc].set(W2).at[:, nc:nc + nd].set(Wd)
    bcat = jnp.zeros((1, d2), b2.dtype).at[0, :nc].set(b2).at[0, nc:nc + nd].set(bd)

    degp_t = _deg_partials(dst, _NP).reshape(_NC, _NP).T
    t1 = _tc_prep1(degp_t, x_pad, W1)
    p = _agg_partials(t1, src, dst, _NP, d).reshape(_NC, _NP, d)
    t2 = _tc_layer2(p, t1, degp_t, b1.reshape(1, d), wcat)
    q = _agg_partials(t2, src, dst, _NP, d2).reshape(_NC, _NP, d2)
    out = _tc_final(q, t2, degp_t, bcat)
    return out[:n, :nc], out[:n, nc:nc + nd]


# speedup vs baseline: 33.7408x; 1.7651x over previous
import functools

import jax
import jax.numpy as jnp
from jax import lax
from jax.experimental import pallas as pl
from jax.experimental.pallas import tpu as pltpu
from jax.experimental.pallas import tpu_sc as plsc

_NC, _NS, _L = 2, 16, 16
_NW = _NC * _NS
_NP = 10240


def _deg_partials(dst, np_pad):
    e = dst.shape[0]
    per_tile = e // _NW
    bsz = 128
    full, tail = divmod(per_tile, bsz)
    rpt = np_pad // _NS
    mesh = plsc.VectorSubcoreMesh(core_axis_name="c", subcore_axis_name="s")

    nbuf = 2
    outer, rem = divmod(full, nbuf)

    @functools.partial(
        pl.kernel,
        out_type=jax.ShapeDtypeStruct((_NC * np_pad,), jnp.float32),
        mesh=mesh,
        scratch_types=[
            [pltpu.VMEM((bsz,), jnp.int32) for _ in range(nbuf)],
            [pltpu.SemaphoreType.DMA for _ in range(nbuf)],
            pltpu.VMEM((bsz,), jnp.float32),
            pltpu.VMEM((tail,), jnp.int32),
            pltpu.VMEM((tail,), jnp.float32),
            pltpu.VMEM((rpt,), jnp.float32),
            pltpu.VMEM_SHARED((np_pad,), jnp.float32),
        ],
    )
    def k(dst_hbm, out_hbm, didx, isem, ones_v, didx_t, ones_t, zeros_v, acc):
        c = lax.axis_index("c")
        s = lax.axis_index("s")
        wid = c * _NS + s
        for j in range(bsz // _L):
            ones_v[pl.ds(j * _L, _L)] = jnp.full((_L,), 1.0, jnp.float32)
        for j in range(tail // _L):
            ones_t[pl.ds(j * _L, _L)] = jnp.full((_L,), 1.0, jnp.float32)
        for j in range(rpt // _L):
            zeros_v[pl.ds(j * _L, _L)] = jnp.zeros((_L,), jnp.float32)
        pltpu.sync_copy(zeros_v, acc.at[pl.ds(s * rpt, rpt)])
        plsc.subcore_barrier()
        base = wid * per_tile

        for b in range(nbuf):
            off = base + b * bsz
            pltpu.async_copy(dst_hbm.at[pl.ds(off, bsz)], didx[b], isem[b])

        def body(g, carry):
            for b in range(nbuf):
                pltpu.make_async_copy(
                    dst_hbm.at[pl.ds(base, bsz)], didx[b], isem[b]).wait()
                pltpu.sync_copy(ones_v, acc.at[didx[b]], add=True)

                @pl.when(g < outer - 1)
                def _():
                    off = base + ((g + 1) * nbuf + b) * bsz
                    pltpu.async_copy(dst_hbm.at[pl.ds(off, bsz)], didx[b],
                                     isem[b])
            return carry

        lax.fori_loop(0, outer, body, 0)
        for i in range(outer * nbuf, full):
            off = base + i * bsz
            pltpu.sync_copy(dst_hbm.at[pl.ds(off, bsz)], didx[0])
            pltpu.sync_copy(ones_v, acc.at[didx[0]], add=True)
        if tail:
            off = base + full * bsz
            pltpu.sync_copy(dst_hbm.at[pl.ds(off, tail)], didx_t)
            pltpu.sync_copy(ones_t, acc.at[didx_t], add=True)
        plsc.subcore_barrier()
        pltpu.sync_copy(acc.at[pl.ds(s * rpt, rpt)],
                        out_hbm.at[pl.ds(c * np_pad + s * rpt, rpt)])

    return k(dst)


def _agg_partials(tbl, src, dst, np_pad, d):
    e = src.shape[0]
    per_tile = e // _NW
    bsz = 128
    full, tail = divmod(per_tile, bsz)
    rpt = np_pad // _NS
    zr = 16
    mesh = plsc.VectorSubcoreMesh(core_axis_name="c", subcore_axis_name="s")

    nbuf = 2
    outer, rem = divmod(full, nbuf)

    @functools.partial(
        pl.kernel,
        out_type=jax.ShapeDtypeStruct((_NC * np_pad, d), jnp.float32),
        mesh=mesh,
        scratch_types=[
            [pltpu.VMEM((bsz,), jnp.int32) for _ in range(nbuf)],
            [pltpu.VMEM((bsz,), jnp.int32) for _ in range(nbuf)],
            [pltpu.VMEM((bsz, d), jnp.float32) for _ in range(nbuf)],
            [pltpu.SemaphoreType.DMA for _ in range(nbuf)],
            [pltpu.SemaphoreType.DMA for _ in range(nbuf)],
            pltpu.VMEM((tail,), jnp.int32),
            pltpu.VMEM((tail,), jnp.int32),
            pltpu.VMEM((tail, d), jnp.float32),
            pltpu.VMEM((zr, d), jnp.float32),
            pltpu.VMEM_SHARED((np_pad, d), jnp.float32),
            pltpu.SemaphoreType.DMA,
        ],
    )
    def k(tbl_hbm, src_hbm, dst_hbm, out_hbm,
          sidx, didx, rows, gsem, isem,
          sidx_t, didx_t, rows_t, zblk, acc, sem):
        c = lax.axis_index("c")
        s = lax.axis_index("s")
        wid = c * _NS + s
        for r in range(zr):
            for j in range(d // _L):
                zblk[r, pl.ds(j * _L, _L)] = jnp.zeros((_L,), jnp.float32)
        for t in range(rpt // zr):
            pltpu.sync_copy(zblk, acc.at[pl.ds(s * rpt + t * zr, zr)])
        plsc.subcore_barrier()
        base = wid * per_tile

        for b in range(nbuf):
            off = base + b * bsz
            pltpu.sync_copy(src_hbm.at[pl.ds(off, bsz)], sidx[b])
            pltpu.async_copy(dst_hbm.at[pl.ds(off, bsz)], didx[b], isem[b])
            pltpu.async_copy(tbl_hbm.at[sidx[b]], rows[b], gsem[b])

        def body(g, carry):
            for b in range(nbuf):
                pltpu.make_async_copy(
                    dst_hbm.at[pl.ds(base, bsz)], didx[b], isem[b]).wait()
                pltpu.make_async_copy(
                    tbl_hbm.at[sidx[b]], rows[b], gsem[b]).wait()
                pltpu.sync_copy(rows[b], acc.at[didx[b]], add=True)

                @pl.when(g < outer - 1)
                def _():
                    off = base + ((g + 1) * nbuf + b) * bsz
                    pltpu.sync_copy(src_hbm.at[pl.ds(off, bsz)], sidx[b])
                    pltpu.async_copy(dst_hbm.at[pl.ds(off, bsz)], didx[b],
                                     isem[b])
                    pltpu.async_copy(tbl_hbm.at[sidx[b]], rows[b], gsem[b])
            return carry

        lax.fori_loop(0, outer, body, 0)
        for i in range(outer * nbuf, full):
            off = base + i * bsz
            pltpu.sync_copy(src_hbm.at[pl.ds(off, bsz)], sidx[0])
            pltpu.sync_copy(dst_hbm.at[pl.ds(off, bsz)], didx[0])
            pltpu.async_copy(tbl_hbm.at[sidx[0]], rows[0], sem).wait()
            pltpu.sync_copy(rows[0], acc.at[didx[0]], add=True)
        if tail:
            off = base + full * bsz
            pltpu.sync_copy(src_hbm.at[pl.ds(off, tail)], sidx_t)
            pltpu.sync_copy(dst_hbm.at[pl.ds(off, tail)], didx_t)
            pltpu.async_copy(tbl_hbm.at[sidx_t], rows_t, sem).wait()
            pltpu.sync_copy(rows_t, acc.at[didx_t], add=True)
        plsc.subcore_barrier()
        pltpu.sync_copy(acc.at[pl.ds(s * rpt, rpt)],
                        out_hbm.at[pl.ds(c * np_pad + s * rpt, rpt)])

    return k(tbl, src, dst)


def _tc_prep1(degp_t, x_pad, w1):
    np_pad, d = x_pad.shape
    blk = 1280
    grid = np_pad // blk

    def body(deg_ref, x_ref, w_ref, o_ref):
        dval = deg_ref[:, 0:1] + deg_ref[:, 1:2] + 1.0
        dis = lax.rsqrt(dval)
        o_ref[...] = jnp.dot(x_ref[...], w_ref[...],
                             preferred_element_type=jnp.float32) * dis

    return pl.pallas_call(
        body,
        grid=(grid,),
        in_specs=[
            pl.BlockSpec((blk, 2), lambda i: (i, 0)),
            pl.BlockSpec((blk, d), lambda i: (i, 0)),
            pl.BlockSpec((d, d), lambda i: (0, 0)),
        ],
        out_specs=pl.BlockSpec((blk, d), lambda i: (i, 0)),
        out_shape=jax.ShapeDtypeStruct((np_pad, d), jnp.float32),
    )(degp_t, x_pad, w1)


def _tc_layer2(p, t1, degp_t, b1, wcat):
    np_pad, d = t1.shape
    d2 = wcat.shape[1]
    blk = 1280
    grid = np_pad // blk

    def body(p_ref, t1_ref, deg_ref, b_ref, w_ref, o_ref):
        dval = deg_ref[:, 0:1] + deg_ref[:, 1:2] + 1.0
        dis = lax.rsqrt(dval)
        agg = p_ref[0] + p_ref[1] + t1_ref[...]
        h = jnp.maximum(agg * dis + b_ref[...], 0.0)
        o_ref[...] = jnp.dot(h, w_ref[...],
                             preferred_element_type=jnp.float32) * dis

    return pl.pallas_call(
        body,
        grid=(grid,),
        in_specs=[
            pl.BlockSpec((2, blk, d), lambda i: (0, i, 0)),
            pl.BlockSpec((blk, d), lambda i: (i, 0)),
            pl.BlockSpec((blk, 2), lambda i: (i, 0)),
            pl.BlockSpec((1, d), lambda i: (0, 0)),
            pl.BlockSpec((d, d2), lambda i: (0, 0)),
        ],
        out_specs=pl.BlockSpec((blk, d2), lambda i: (i, 0)),
        out_shape=jax.ShapeDtypeStruct((np_pad, d2), jnp.float32),
    )(p, t1, degp_t, b1, wcat)


def _tc_final(q, t2, degp_t, bcat):
    np_pad, d2 = t2.shape
    blk = 1280
    grid = np_pad // blk

    def body(q_ref, t2_ref, deg_ref, b_ref, o_ref):
        dval = deg_ref[:, 0:1] + deg_ref[:, 1:2] + 1.0
        dis = lax.rsqrt(dval)
        o_ref[...] = (q_ref[0] + q_ref[1] + t2_ref[...]) * dis + b_ref[...]

    return pl.pallas_call(
        body,
        grid=(grid,),
        in_specs=[
            pl.BlockSpec((2, blk, d2), lambda i: (0, i, 0)),
            pl.BlockSpec((blk, d2), lambda i: (i, 0)),
            pl.BlockSpec((blk, 2), lambda i: (i, 0)),
            pl.BlockSpec((1, d2), lambda i: (0, 0)),
        ],
        out_specs=pl.BlockSpec((blk, d2), lambda i: (i, 0)),
        out_shape=jax.ShapeDtypeStruct((np_pad, d2), jnp.float32),
    )(q, t2, degp_t, bcat)


def kernel(x, edge_index, W1, b1, W2, b2, Wd, bd):
    n, d = x.shape
    src = edge_index[0]
    dst = edge_index[1]
    nc = W2.shape[1]
    nd = Wd.shape[1]
    d2 = 128

    x_pad = jnp.zeros((_NP, d), x.dtype).at[:n].set(x)
    wcat = jnp.zeros((d, d2), W2.dtype).at[:, :nc].set(W2).at[:, nc:nc + nd].set(Wd)
    bcat = jnp.zeros((1, d2), b2.dtype).at[0, :nc].set(b2).at[0, nc:nc + nd].set(bd)

    degp_t = _deg_partials(dst, _NP).reshape(_NC, _NP).T
    t1 = _tc_prep1(degp_t, x_pad, W1)
    p = _agg_partials(t1, src, dst, _NP, d).reshape(_NC, _NP, d)
    t2 = _tc_layer2(p, t1, degp_t, b1.reshape(1, d), wcat)
    q = _agg_partials(t2, src, dst, _NP, d2).reshape(_NC, _NP, d2)
    out = _tc_final(q, t2, degp_t, bcat)
    return out[:n, :nc], out[:n, nc:nc + nd]
